# baseline (speedup 1.0000x reference)
import functools

import jax
import jax.numpy as jnp
from jax import lax
from jax.experimental import pallas as pl
from jax.experimental.pallas import tpu as pltpu
import jax.experimental.pallas.tpu_sc as plsc

N = 10000
D = 128
E = 320000
G = 16
C = 2

NC = 2
NS = 16
NW = NC * NS
K = 128
CH = 80
EP = NW * K * CH
NB = 2
NP = 10240
RPT = NP // NS

BLK = 512
NBLK = NP // BLK


def _sc_segment_body(h_hbm, pidx_hbm, out_hbm, pidx, sbuf, dbuf, rows, acc,
                     sem_g):
    c = lax.axis_index("c")
    s = lax.axis_index("s")
    w = c * NS + s
    pltpu.sync_copy(h_hbm.at[pl.ds(s * RPT, RPT)], acc.at[pl.ds(s * RPT, RPT)])
    pltpu.sync_copy(pidx_hbm.at[w], pidx)
    plsc.subcore_barrier()

    def unpack(j, b):
        for q in range(K // 16):
            v = pidx[j, pl.ds(q * 16, 16)]
            sbuf[b, pl.ds(q * 16, 16)] = v & 0x3FFF
            dbuf[b, pl.ds(q * 16, 16)] = v >> 14

    def g_start(b):
        pltpu.async_copy(h_hbm.at[sbuf.at[b]], rows.at[b], sem_g)

    def g_wait(b):
        pltpu.make_async_copy(h_hbm.at[pl.ds(0, K)], rows.at[b], sem_g).wait()

    def s_add(b):
        pltpu.sync_copy(rows.at[b], acc.at[dbuf.at[b]], add=True)

    unpack(0, 0)
    g_start(0)

    def pair(g, carry):
        j = 2 * g
        unpack(j + 1, 1); g_start(1)
        g_wait(0); s_add(0)
        unpack(j + 2, 0); g_start(0)
        g_wait(1); s_add(1)
        return carry

    lax.fori_loop(0, (CH - 2) // 2, pair, 0)
    unpack(CH - 1, 1); g_start(1)
    g_wait(0); s_add(0)
    g_wait(1); s_add(1)
    plsc.subcore_barrier()
    pltpu.sync_copy(acc.at[pl.ds(s * RPT, RPT)],
                    out_hbm.at[c, pl.ds(s * RPT, RPT)])


@functools.cache
def _sc_segment():
    return pl.kernel(
        _sc_segment_body,
        out_type=jax.ShapeDtypeStruct((NC, NP, D), jnp.float32),
        mesh=plsc.VectorSubcoreMesh(core_axis_name="c", subcore_axis_name="s",
                                    num_cores=NC, num_subcores=NS),
        scratch_types=[
            pltpu.VMEM((CH, K), jnp.int32),
            pltpu.VMEM((2, K), jnp.int32),
            pltpu.VMEM((2, K), jnp.int32),
            pltpu.VMEM((NB, K, D), jnp.float32),
            pltpu.VMEM_SHARED((NP, D), jnp.float32),
            pltpu.SemaphoreType.DMA,
        ],
    )


def _mlp_body(p0_ref, p1_ref, h_ref, w0_ref, b0_ref, w1_ref, b1_ref, o_ref):
    i = pl.program_id(0)
    t = p0_ref[...] + p1_ref[...] - h_ref[...]
    a = jnp.dot(t, w0_ref[...], preferred_element_type=jnp.float32) + b0_ref[...]
    a = jnp.maximum(a, 0.0)
    a = jnp.dot(a, w1_ref[...], preferred_element_type=jnp.float32) + b1_ref[...]
    a = jnp.maximum(a, 0.0)
    rows = i * BLK + lax.broadcasted_iota(jnp.int32, (BLK, 1), 0)
    o_ref[...] = jnp.where(rows < N, a, 0.0)


def _tc_mlp(p0, p1, h, w0, b0, w1, b1):
    row_spec = pl.BlockSpec((BLK, D), lambda i: (i, 0))
    full = lambda shape: pl.BlockSpec(shape, lambda i: (0, 0))
    return pl.pallas_call(
        _mlp_body,
        grid=(NBLK,),
        in_specs=[row_spec, row_spec, row_spec,
                  full((D, D)), full((1, D)), full((D, D)), full((1, D))],
        out_specs=row_spec,
        out_shape=jax.ShapeDtypeStruct((NP, D), jnp.float32),
    )(p0, p1, h, w0, b0, w1, b1)


def _final_body(p0_ref, p1_ref, h_ref, batch_ref, w0_ref, b0_ref, w1_ref,
                b1_ref, wout_ref, bout_ref, o_ref, pooled):
    i = pl.program_id(0)
    t = p0_ref[...] + p1_ref[...] - h_ref[...]
    a = jnp.dot(t, w0_ref[...], preferred_element_type=jnp.float32) + b0_ref[...]
    a = jnp.maximum(a, 0.0)
    a = jnp.dot(a, w1_ref[...], preferred_element_type=jnp.float32) + b1_ref[...]
    a = jnp.maximum(a, 0.0)

    @pl.when(i == 0)
    def _():
        pooled[...] = jnp.full((G, D), -jnp.inf, dtype=jnp.float32)

    b = batch_ref[...]
    for g in range(G):
        v = jnp.where(b == g, a, -jnp.inf).max(axis=0)
        pooled[g, :] = jnp.maximum(pooled[g, :], v)

    @pl.when(i == NBLK - 1)
    def _():
        o_ref[...] = (
            jnp.dot(pooled[...], wout_ref[...], preferred_element_type=jnp.float32)
            + bout_ref[...])


def _tc_final(p0, p1, h, batch2d, w0, b0, w1, b1, wout, bout):
    row_spec = pl.BlockSpec((BLK, D), lambda i: (i, 0))
    full = lambda shape: pl.BlockSpec(shape, lambda i: (0, 0))
    return pl.pallas_call(
        _final_body,
        grid=(NBLK,),
        in_specs=[row_spec, row_spec, row_spec,
                  pl.BlockSpec((BLK, 1), lambda i: (i, 0)),
                  full((D, D)), full((1, D)), full((D, D)), full((1, D)),
                  full((D, C)), full((1, C))],
        out_specs=full((G, C)),
        out_shape=jax.ShapeDtypeStruct((G, C), jnp.float32),
        scratch_shapes=[pltpu.VMEM((G, D), jnp.float32)],
    )(p0, p1, h, batch2d, w0, b0, w1, b1, wout, bout)


def kernel(x, edge_index, batch, W00, b00, W01, b01, W10, b10, W11, b11,
           W20, b20, W21, b21, W30, b30, W31, b31, Wout, bout):
    xp = jnp.zeros((NP, D), jnp.float32).at[:N].set(x)
    npad = EP - E
    srcp = jnp.concatenate([edge_index[0], jnp.full((npad,), N, jnp.int32)])
    dstp = jnp.concatenate(
        [edge_index[1], (jnp.arange(npad, dtype=jnp.int32) * 131) % N])
    pidx = ((dstp << 14) | srcp).reshape(NW, CH, K)
    batch2d = jnp.full((NP, 1), G, jnp.int32).at[:N, 0].set(batch)

    ws = [(W00, b00.reshape(1, D), W01, b01.reshape(1, D)),
          (W10, b10.reshape(1, D), W11, b11.reshape(1, D)),
          (W20, b20.reshape(1, D), W21, b21.reshape(1, D)),
          (W30, b30.reshape(1, D), W31, b31.reshape(1, D))]

    seg = _sc_segment()
    h = xp
    for l in range(3):
        p = seg(h, pidx)
        h = _tc_mlp(p[0], p[1], h, *ws[l])
    p = seg(h, pidx)
    return _tc_final(p[0], p[1], h, batch2d, *ws[3],
                     Wout, bout.reshape(1, C))


# speedup vs baseline: 3.6471x; 3.6471x over previous
import functools

import jax
import jax.numpy as jnp
from jax import lax
from jax.experimental import pallas as pl
from jax.experimental.pallas import tpu as pltpu
import jax.experimental.pallas.tpu_sc as plsc

N = 10000
D = 128
E = 320000
G = 16
C = 2

NC = 2
NS = 16
NW = NC * NS
K = 128
CH = 80
EP = NW * K * CH
NB = 2
NP = 10240
RPT = NP // NS

BLK = 512
NBLK = NP // BLK


def _sc_segment_body(h_hbm, pidx_hbm, out_hbm, pidx, sbuf, dbuf, rows, acc,
                     sem_g):
    c = lax.axis_index("c")
    s = lax.axis_index("s")
    w = c * NS + s
    pltpu.sync_copy(h_hbm.at[pl.ds(s * RPT, RPT)], acc.at[pl.ds(s * RPT, RPT)])
    pltpu.sync_copy(pidx_hbm.at[w], pidx)
    plsc.subcore_barrier()

    def unpack(j, b):
        for q in range(K // 16):
            v = pidx[j, pl.ds(q * 16, 16)]
            sbuf[b, pl.ds(q * 16, 16)] = v & 0x3FFF
            dbuf[b, pl.ds(q * 16, 16)] = v >> 14

    def g_start(b):
        pltpu.async_copy(h_hbm.at[sbuf.at[b]], rows.at[b], sem_g)

    def g_wait(b):
        pltpu.make_async_copy(h_hbm.at[pl.ds(0, K)], rows.at[b], sem_g).wait()

    def s_add(b):
        pltpu.sync_copy(rows.at[b], acc.at[dbuf.at[b]], add=True)

    unpack(0, 0)
    g_start(0)

    def pair(g, carry):
        j = 2 * g
        unpack(j + 1, 1); g_start(1)
        g_wait(0); s_add(0)
        unpack(j + 2, 0); g_start(0)
        g_wait(1); s_add(1)
        return carry

    lax.fori_loop(0, (CH - 2) // 2, pair, 0)
    unpack(CH - 1, 1); g_start(1)
    g_wait(0); s_add(0)
    g_wait(1); s_add(1)
    plsc.subcore_barrier()
    pltpu.sync_copy(acc.at[pl.ds(s * RPT, RPT)],
                    out_hbm.at[c, pl.ds(s * RPT, RPT)])


@functools.cache
def _sc_segment():
    return pl.kernel(
        _sc_segment_body,
        out_type=jax.ShapeDtypeStruct((NC, NP, D), jnp.float32),
        mesh=plsc.VectorSubcoreMesh(core_axis_name="c", subcore_axis_name="s",
                                    num_cores=NC, num_subcores=NS),
        scratch_types=[
            pltpu.VMEM((CH, K), jnp.int32),
            pltpu.VMEM((2, K), jnp.int32),
            pltpu.VMEM((2, K), jnp.int32),
            pltpu.VMEM((NB, K, D), jnp.float32),
            pltpu.VMEM_SHARED((NP, D), jnp.float32),
            pltpu.SemaphoreType.DMA,
        ],
    )


def _mlp_body(p0_ref, p1_ref, h_ref, w0_ref, b0_ref, w1_ref, b1_ref, o_ref):
    i = pl.program_id(0)
    t = p0_ref[...] + p1_ref[...] - h_ref[...]
    a = jnp.dot(t, w0_ref[...], preferred_element_type=jnp.float32) + b0_ref[...]
    a = jnp.maximum(a, 0.0)
    a = jnp.dot(a, w1_ref[...], preferred_element_type=jnp.float32) + b1_ref[...]
    a = jnp.maximum(a, 0.0)
    rows = i * BLK + lax.broadcasted_iota(jnp.int32, (BLK, 1), 0)
    o_ref[...] = jnp.where(rows < N, a, 0.0)


def _tc_mlp(p0, p1, h, w0, b0, w1, b1):
    row_spec = pl.BlockSpec((BLK, D), lambda i: (i, 0))
    full = lambda shape: pl.BlockSpec(shape, lambda i: (0, 0))
    return pl.pallas_call(
        _mlp_body,
        grid=(NBLK,),
        in_specs=[row_spec, row_spec, row_spec,
                  full((D, D)), full((1, D)), full((D, D)), full((1, D))],
        out_specs=row_spec,
        out_shape=jax.ShapeDtypeStruct((NP, D), jnp.float32),
    )(p0, p1, h, w0, b0, w1, b1)


def _final_body(p0_ref, p1_ref, h_ref, batch_ref, w0_ref, b0_ref, w1_ref,
                b1_ref, wout_ref, bout_ref, o_ref, pooled):
    i = pl.program_id(0)
    t = p0_ref[...] + p1_ref[...] - h_ref[...]
    a = jnp.dot(t, w0_ref[...], preferred_element_type=jnp.float32) + b0_ref[...]
    a = jnp.maximum(a, 0.0)
    a = jnp.dot(a, w1_ref[...], preferred_element_type=jnp.float32) + b1_ref[...]
    a = jnp.maximum(a, 0.0)

    @pl.when(i == 0)
    def _():
        pooled[...] = jnp.full((G, D), -jnp.inf, dtype=jnp.float32)

    b = batch_ref[...]
    for g in range(G):
        v = jnp.where(b == g, a, -jnp.inf).max(axis=0)
        pooled[g, :] = jnp.maximum(pooled[g, :], v)

    @pl.when(i == NBLK - 1)
    def _():
        o_ref[...] = (
            jnp.dot(pooled[...], wout_ref[...], preferred_element_type=jnp.float32)
            + bout_ref[...])


def _tc_final(p0, p1, h, batch2d, w0, b0, w1, b1, wout, bout):
    row_spec = pl.BlockSpec((BLK, D), lambda i: (i, 0))
    full = lambda shape: pl.BlockSpec(shape, lambda i: (0, 0))
    return pl.pallas_call(
        _final_body,
        grid=(NBLK,),
        in_specs=[row_spec, row_spec, row_spec,
                  pl.BlockSpec((BLK, 1), lambda i: (i, 0)),
                  full((D, D)), full((1, D)), full((D, D)), full((1, D)),
                  full((D, C)), full((1, C))],
        out_specs=full((G, C)),
        out_shape=jax.ShapeDtypeStruct((G, C), jnp.float32),
        scratch_shapes=[pltpu.VMEM((G, D), jnp.float32)],
    )(p0, p1, h, batch2d, w0, b0, w1, b1, wout, bout)


def kernel(x, edge_index, batch, W00, b00, W01, b01, W10, b10, W11, b11,
           W20, b20, W21, b21, W30, b30, W31, b31, Wout, bout):
    xp = jnp.zeros((NP, D), jnp.float32).at[:N].set(x)
    npad = EP - E
    srcp = jnp.concatenate(
        [edge_index[0], N + (jnp.arange(npad, dtype=jnp.int32) % (NP - N))])
    dstp = jnp.concatenate(
        [edge_index[1], (jnp.arange(npad, dtype=jnp.int32) * 131) % N])
    pidx = ((dstp << 14) | srcp).reshape(NW, CH, K)
    batch2d = jnp.full((NP, 1), G, jnp.int32).at[:N, 0].set(batch)

    ws = [(W00, b00.reshape(1, D), W01, b01.reshape(1, D)),
          (W10, b10.reshape(1, D), W11, b11.reshape(1, D)),
          (W20, b20.reshape(1, D), W21, b21.reshape(1, D)),
          (W30, b30.reshape(1, D), W31, b31.reshape(1, D))]

    seg = _sc_segment()
    h = xp
    for l in range(3):
        p = seg(h, pidx)
        h = _tc_mlp(p[0], p[1], h, *ws[l])
    p = seg(h, pidx)
    return _tc_final(p[0], p[1], h, batch2d, *ws[3],
                     Wout, bout.reshape(1, C))


# speedup vs baseline: 3.7378x; 1.0249x over previous
import functools

import jax
import jax.numpy as jnp
from jax import lax
from jax.experimental import pallas as pl
from jax.experimental.pallas import tpu as pltpu
import jax.experimental.pallas.tpu_sc as plsc

N = 10000
D = 128
E = 320000
G = 16
C = 2

NC = 2
NS = 16
NW = NC * NS
K = 64
CH = 160
EP = NW * K * CH
NB = 4
NP = 10240
RPT = NP // NS

BLK = 512
NBLK = NP // BLK


def _sc_segment_body(h_hbm, pidx_hbm, out_hbm, pidx, sbuf, dbuf, rows, acc,
                     sem_g, sem_s):
    c = lax.axis_index("c")
    s = lax.axis_index("s")
    w = c * NS + s
    pltpu.sync_copy(h_hbm.at[pl.ds(s * RPT, RPT)], acc.at[pl.ds(s * RPT, RPT)])
    pltpu.sync_copy(pidx_hbm.at[w], pidx)
    plsc.subcore_barrier()

    def unpack(row, half, b):
        for q in range(K // 16):
            v = pidx[row, pl.ds(half * K + q * 16, 16)]
            sbuf[b, pl.ds(q * 16, 16)] = v & 0x3FFF
            dbuf[b, pl.ds(q * 16, 16)] = v >> 14

    def g_start(b):
        pltpu.async_copy(h_hbm.at[sbuf.at[b]], rows.at[b], sem_g)

    def g_wait(b):
        pltpu.make_async_copy(h_hbm.at[pl.ds(0, K)], rows.at[b], sem_g).wait()

    def s_start(b):
        pltpu.async_copy(rows.at[b], acc.at[dbuf.at[b]], sem_s, add=True)

    def s_drain(b):
        pltpu.make_async_copy(rows.at[b], out_hbm.at[c, pl.ds(0, K)],
                              sem_s).wait()

    unpack(0, 0, 0); g_start(0)
    unpack(0, 1, 1); g_start(1)
    g_wait(0); s_start(0); unpack(1, 0, 2); g_start(2)
    g_wait(1); s_start(1); unpack(1, 1, 3); g_start(3)

    def group(g, carry):
        for b in range(4):
            bj = (b + 2) % 4
            g_wait(bj)
            s_start(bj)
            s_drain(b)
            unpack(2 * g + 2 + b // 2, b % 2, b)
            g_start(b)
        return carry

    lax.fori_loop(0, (CH - 4) // 4, group, 0)
    g_wait(2); s_start(2); s_drain(0)
    g_wait(3); s_start(3); s_drain(1)
    s_drain(2)
    s_drain(3)
    plsc.subcore_barrier()
    pltpu.sync_copy(acc.at[pl.ds(s * RPT, RPT)],
                    out_hbm.at[c, pl.ds(s * RPT, RPT)])


@functools.cache
def _sc_segment():
    return pl.kernel(
        _sc_segment_body,
        out_type=jax.ShapeDtypeStruct((NC, NP, D), jnp.float32),
        mesh=plsc.VectorSubcoreMesh(core_axis_name="c", subcore_axis_name="s",
                                    num_cores=NC, num_subcores=NS),
        scratch_types=[
            pltpu.VMEM((CH // 2, 2 * K), jnp.int32),
            pltpu.VMEM((NB, K), jnp.int32),
            pltpu.VMEM((NB, K), jnp.int32),
            pltpu.VMEM((NB, K, D), jnp.float32),
            pltpu.VMEM_SHARED((NP, D), jnp.float32),
            pltpu.SemaphoreType.DMA,
            pltpu.SemaphoreType.DMA,
        ],
    )


def _mlp_body(p0_ref, p1_ref, h_ref, w0_ref, b0_ref, w1_ref, b1_ref, o_ref):
    i = pl.program_id(0)
    t = p0_ref[0] + p1_ref[0] - h_ref[...]
    a = jnp.dot(t, w0_ref[...], preferred_element_type=jnp.float32) + b0_ref[...]
    a = jnp.maximum(a, 0.0)
    a = jnp.dot(a, w1_ref[...], preferred_element_type=jnp.float32) + b1_ref[...]
    a = jnp.maximum(a, 0.0)
    rows = i * BLK + lax.broadcasted_iota(jnp.int32, (BLK, 1), 0)
    o_ref[...] = jnp.where(rows < N, a, 0.0)


def _tc_mlp(p, h, w0, b0, w1, b1):
    row_spec = pl.BlockSpec((BLK, D), lambda i: (i, 0))
    full = lambda shape: pl.BlockSpec(shape, lambda i: (0, 0))
    return pl.pallas_call(
        _mlp_body,
        grid=(NBLK,),
        in_specs=[pl.BlockSpec((1, BLK, D), lambda i: (0, i, 0)),
                  pl.BlockSpec((1, BLK, D), lambda i: (1, i, 0)),
                  row_spec,
                  full((D, D)), full((1, D)), full((D, D)), full((1, D))],
        out_specs=row_spec,
        out_shape=jax.ShapeDtypeStruct((NP, D), jnp.float32),
    )(p, p, h, w0, b0, w1, b1)


def _final_body(p0_ref, p1_ref, h_ref, batch_ref, w0_ref, b0_ref, w1_ref,
                b1_ref, wout_ref, bout_ref, o_ref, pooled):
    i = pl.program_id(0)
    t = p0_ref[0] + p1_ref[0] - h_ref[...]
    a = jnp.dot(t, w0_ref[...], preferred_element_type=jnp.float32) + b0_ref[...]
    a = jnp.maximum(a, 0.0)
    a = jnp.dot(a, w1_ref[...], preferred_element_type=jnp.float32) + b1_ref[...]
    a = jnp.maximum(a, 0.0)

    @pl.when(i == 0)
    def _():
        pooled[...] = jnp.full((G, D), -jnp.inf, dtype=jnp.float32)

    b = batch_ref[...]
    for g in range(G):
        v = jnp.where(b == g, a, -jnp.inf).max(axis=0)
        pooled[g, :] = jnp.maximum(pooled[g, :], v)

    @pl.when(i == NBLK - 1)
    def _():
        o_ref[...] = (
            jnp.dot(pooled[...], wout_ref[...], preferred_element_type=jnp.float32)
            + bout_ref[...])


def _tc_final(p, h, batch2d, w0, b0, w1, b1, wout, bout):
    row_spec = pl.BlockSpec((BLK, D), lambda i: (i, 0))
    full = lambda shape: pl.BlockSpec(shape, lambda i: (0, 0))
    return pl.pallas_call(
        _final_body,
        grid=(NBLK,),
        in_specs=[pl.BlockSpec((1, BLK, D), lambda i: (0, i, 0)),
                  pl.BlockSpec((1, BLK, D), lambda i: (1, i, 0)),
                  row_spec,
                  pl.BlockSpec((BLK, 1), lambda i: (i, 0)),
                  full((D, D)), full((1, D)), full((D, D)), full((1, D)),
                  full((D, C)), full((1, C))],
        out_specs=full((G, C)),
        out_shape=jax.ShapeDtypeStruct((G, C), jnp.float32),
        scratch_shapes=[pltpu.VMEM((G, D), jnp.float32)],
    )(p, p, h, batch2d, w0, b0, w1, b1, wout, bout)


def kernel(x, edge_index, batch, W00, b00, W01, b01, W10, b10, W11, b11,
           W20, b20, W21, b21, W30, b30, W31, b31, Wout, bout):
    xp = jnp.zeros((NP, D), jnp.float32).at[:N].set(x)
    npad = EP - E
    srcp = jnp.concatenate(
        [edge_index[0], N + (jnp.arange(npad, dtype=jnp.int32) % (NP - N))])
    dstp = jnp.concatenate(
        [edge_index[1], (jnp.arange(npad, dtype=jnp.int32) * 131) % N])
    pidx = ((dstp << 14) | srcp).reshape(NW, CH // 2, 2 * K)
    batch2d = jnp.full((NP, 1), G, jnp.int32).at[:N, 0].set(batch)

    ws = [(W00, b00.reshape(1, D), W01, b01.reshape(1, D)),
          (W10, b10.reshape(1, D), W11, b11.reshape(1, D)),
          (W20, b20.reshape(1, D), W21, b21.reshape(1, D)),
          (W30, b30.reshape(1, D), W31, b31.reshape(1, D))]

    seg = _sc_segment()
    h = xp
    for l in range(3):
        p = seg(h, pidx)
        h = _tc_mlp(p, h, *ws[l])
    p = seg(h, pidx)
    return _tc_final(p, h, batch2d, *ws[3], Wout, bout.reshape(1, C))


# speedup vs baseline: 3.7835x; 1.0122x over previous
import functools

import jax
import jax.numpy as jnp
from jax import lax
from jax.experimental import pallas as pl
from jax.experimental.pallas import tpu as pltpu
import jax.experimental.pallas.tpu_sc as plsc

N = 10000
D = 128
E = 320000
G = 16
C = 2

NC = 2
NS = 16
NW = NC * NS
K = 64
CH = 160
EP = NW * K * CH
NB = 4
NP = 10240
RPT = NP // NS

BLK = 512
NBLK = NP // BLK


def _sc_segment_body(h_hbm, zrows_hbm, pidx_hbm, out_hbm, pidx, sbuf, dbuf,
                     rows, acc, sem_g, sem_s, sem_i):
    c = lax.axis_index("c")
    s = lax.axis_index("s")
    w = c * NS + s
    pltpu.sync_copy(pidx_hbm.at[w], pidx)
    @pl.when(c == 0)
    def _():
        pltpu.async_copy(h_hbm.at[pl.ds(s * RPT, RPT)],
                         acc.at[pl.ds(s * RPT, RPT)], sem_i)

    @pl.when(c == 1)
    def _():
        pltpu.async_copy(zrows_hbm, acc.at[pl.ds(s * RPT, RPT)], sem_i)

    def unpack(row, half, b):
        for q in range(K // 16):
            v = pidx[row, pl.ds(half * K + q * 16, 16)]
            sbuf[b, pl.ds(q * 16, 16)] = v & 0x3FFF
            dbuf[b, pl.ds(q * 16, 16)] = v >> 14

    def g_start(b):
        pltpu.async_copy(h_hbm.at[sbuf.at[b]], rows.at[b], sem_g)

    def g_wait(b):
        pltpu.make_async_copy(h_hbm.at[pl.ds(0, K)], rows.at[b], sem_g).wait()

    def s_start(b):
        pltpu.async_copy(rows.at[b], acc.at[dbuf.at[b]], sem_s, add=True)

    def s_drain(b):
        pltpu.make_async_copy(rows.at[b], out_hbm.at[c, pl.ds(0, K)],
                              sem_s).wait()

    unpack(0, 0, 0); g_start(0)
    unpack(0, 1, 1); g_start(1)
    unpack(1, 0, 2); g_start(2)
    unpack(1, 1, 3); g_start(3)
    pltpu.make_async_copy(zrows_hbm, acc.at[pl.ds(s * RPT, RPT)], sem_i).wait()
    plsc.subcore_barrier()
    g_wait(0); s_start(0)
    g_wait(1); s_start(1)

    def group(g, carry):
        for b in range(4):
            bj = (b + 2) % 4
            g_wait(bj)
            s_start(bj)
            s_drain(b)
            unpack(2 * g + 2 + b // 2, b % 2, b)
            g_start(b)
        return carry

    lax.fori_loop(0, (CH - 4) // 4, group, 0)
    g_wait(2); s_start(2); s_drain(0)
    g_wait(3); s_start(3); s_drain(1)
    s_drain(2)
    s_drain(3)
    plsc.subcore_barrier()
    pltpu.sync_copy(acc.at[pl.ds(s * RPT, RPT)],
                    out_hbm.at[c, pl.ds(s * RPT, RPT)])


@functools.cache
def _sc_segment():
    return pl.kernel(
        _sc_segment_body,
        out_type=jax.ShapeDtypeStruct((NC, NP, D), jnp.float32),
        mesh=plsc.VectorSubcoreMesh(core_axis_name="c", subcore_axis_name="s",
                                    num_cores=NC, num_subcores=NS),
        scratch_types=[
            pltpu.VMEM((CH // 2, 2 * K), jnp.int32),
            pltpu.VMEM((NB, K), jnp.int32),
            pltpu.VMEM((NB, K), jnp.int32),
            pltpu.VMEM((NB, K, D), jnp.float32),
            pltpu.VMEM_SHARED((NP, D), jnp.float32),
            pltpu.SemaphoreType.DMA,
            pltpu.SemaphoreType.DMA,
            pltpu.SemaphoreType.DMA,
        ],
    )


def _mlp_body(p0_ref, p1_ref, w0_ref, b0_ref, w1_ref, b1_ref, o_ref):
    i = pl.program_id(0)
    t = p0_ref[0] + p1_ref[0]
    a = jnp.dot(t, w0_ref[...], preferred_element_type=jnp.float32) + b0_ref[...]
    a = jnp.maximum(a, 0.0)
    a = jnp.dot(a, w1_ref[...], preferred_element_type=jnp.float32) + b1_ref[...]
    a = jnp.maximum(a, 0.0)
    rows = i * BLK + lax.broadcasted_iota(jnp.int32, (BLK, 1), 0)
    o_ref[...] = jnp.where(rows < N, a, 0.0)


def _tc_mlp(p, w0, b0, w1, b1):
    row_spec = pl.BlockSpec((BLK, D), lambda i: (i, 0))
    full = lambda shape: pl.BlockSpec(shape, lambda i: (0, 0))
    return pl.pallas_call(
        _mlp_body,
        grid=(NBLK,),
        in_specs=[pl.BlockSpec((1, BLK, D), lambda i: (0, i, 0)),
                  pl.BlockSpec((1, BLK, D), lambda i: (1, i, 0)),
                  full((D, D)), full((1, D)), full((D, D)), full((1, D))],
        out_specs=row_spec,
        out_shape=jax.ShapeDtypeStruct((NP, D), jnp.float32),
    )(p, p, w0, b0, w1, b1)


def _final_body(p0_ref, p1_ref, batch_ref, w0_ref, b0_ref, w1_ref,
                b1_ref, wout_ref, bout_ref, o_ref, pooled):
    i = pl.program_id(0)
    t = p0_ref[0] + p1_ref[0]
    a = jnp.dot(t, w0_ref[...], preferred_element_type=jnp.float32) + b0_ref[...]
    a = jnp.maximum(a, 0.0)
    a = jnp.dot(a, w1_ref[...], preferred_element_type=jnp.float32) + b1_ref[...]
    a = jnp.maximum(a, 0.0)

    @pl.when(i == 0)
    def _():
        pooled[...] = jnp.full((G, D), -jnp.inf, dtype=jnp.float32)

    b = batch_ref[...]
    for g in range(G):
        v = jnp.where(b == g, a, -jnp.inf).max(axis=0)
        pooled[g, :] = jnp.maximum(pooled[g, :], v)

    @pl.when(i == NBLK - 1)
    def _():
        o_ref[...] = (
            jnp.dot(pooled[...], wout_ref[...], preferred_element_type=jnp.float32)
            + bout_ref[...])


def _tc_final(p, batch2d, w0, b0, w1, b1, wout, bout):
    full = lambda shape: pl.BlockSpec(shape, lambda i: (0, 0))
    return pl.pallas_call(
        _final_body,
        grid=(NBLK,),
        in_specs=[pl.BlockSpec((1, BLK, D), lambda i: (0, i, 0)),
                  pl.BlockSpec((1, BLK, D), lambda i: (1, i, 0)),
                  pl.BlockSpec((BLK, 1), lambda i: (i, 0)),
                  full((D, D)), full((1, D)), full((D, D)), full((1, D)),
                  full((D, C)), full((1, C))],
        out_specs=full((G, C)),
        out_shape=jax.ShapeDtypeStruct((G, C), jnp.float32),
        scratch_shapes=[pltpu.VMEM((G, D), jnp.float32)],
    )(p, p, batch2d, w0, b0, w1, b1, wout, bout)


def kernel(x, edge_index, batch, W00, b00, W01, b01, W10, b10, W11, b11,
           W20, b20, W21, b21, W30, b30, W31, b31, Wout, bout):
    xp = jnp.zeros((NP, D), jnp.float32).at[:N].set(x)
    npad = EP - E
    srcp = jnp.concatenate(
        [edge_index[0], N + (jnp.arange(npad, dtype=jnp.int32) % (NP - N))])
    dstp = jnp.concatenate(
        [edge_index[1], (jnp.arange(npad, dtype=jnp.int32) * 131) % N])
    pidx = ((dstp << 14) | srcp).reshape(NW, CH // 2, 2 * K)
    batch2d = jnp.full((NP, 1), G, jnp.int32).at[:N, 0].set(batch)

    ws = [(W00, b00.reshape(1, D), W01, b01.reshape(1, D)),
          (W10, b10.reshape(1, D), W11, b11.reshape(1, D)),
          (W20, b20.reshape(1, D), W21, b21.reshape(1, D)),
          (W30, b30.reshape(1, D), W31, b31.reshape(1, D))]

    zrows = jnp.zeros((RPT, D), jnp.float32)
    seg = _sc_segment()
    h = xp
    for l in range(3):
        p = seg(h, zrows, pidx)
        h = _tc_mlp(p, *ws[l])
    p = seg(h, zrows, pidx)
    return _tc_final(p, batch2d, *ws[3], Wout, bout.reshape(1, C))


# speedup vs baseline: 3.9246x; 1.0373x over previous
import functools

import jax
import jax.numpy as jnp
from jax import lax
from jax.experimental import pallas as pl
from jax.experimental.pallas import tpu as pltpu
import jax.experimental.pallas.tpu_sc as plsc

N = 10000
D = 128
E = 320000
G = 16
C = 2

NC = 2
NS = 16
NW = NC * NS
K = 64
CH = 160
EP = NW * K * CH
NB = 4
NP = 10240
RPT = NP // NS

BLK = 1024
NBLK = NP // BLK


def _sc_segment_body(h_hbm, zrows_hbm, pidx_hbm, out_hbm, pidx, sbuf, dbuf,
                     rows, acc, sem_g, sem_s, sem_i):
    c = lax.axis_index("c")
    s = lax.axis_index("s")
    w = c * NS + s
    pltpu.sync_copy(pidx_hbm.at[w], pidx)
    @pl.when(c == 0)
    def _():
        pltpu.async_copy(h_hbm.at[pl.ds(s * RPT, RPT)],
                         acc.at[pl.ds(s * RPT, RPT)], sem_i)

    @pl.when(c == 1)
    def _():
        pltpu.async_copy(zrows_hbm, acc.at[pl.ds(s * RPT, RPT)], sem_i)

    def unpack(row, half, b):
        for q in range(K // 16):
            v = pidx[row, pl.ds(half * K + q * 16, 16)]
            sbuf[b, pl.ds(q * 16, 16)] = v & 0x3FFF
            dbuf[b, pl.ds(q * 16, 16)] = v >> 14

    def g_start(b):
        pltpu.async_copy(h_hbm.at[sbuf.at[b]], rows.at[b], sem_g)

    def g_wait(b):
        pltpu.make_async_copy(h_hbm.at[pl.ds(0, K)], rows.at[b], sem_g).wait()

    def s_start(b):
        pltpu.async_copy(rows.at[b], acc.at[dbuf.at[b]], sem_s, add=True)

    def s_drain(b):
        pltpu.make_async_copy(rows.at[b], out_hbm.at[c, pl.ds(0, K)],
                              sem_s).wait()

    unpack(0, 0, 0); g_start(0)
    unpack(0, 1, 1); g_start(1)
    unpack(1, 0, 2); g_start(2)
    unpack(1, 1, 3); g_start(3)
    pltpu.make_async_copy(zrows_hbm, acc.at[pl.ds(s * RPT, RPT)], sem_i).wait()
    plsc.subcore_barrier()
    g_wait(0); s_start(0)
    g_wait(1); s_start(1)

    def group(g, carry):
        for b in range(4):
            bj = (b + 2) % 4
            g_wait(bj)
            s_start(bj)
            s_drain(b)
            unpack(2 * g + 2 + b // 2, b % 2, b)
            g_start(b)
        return carry

    lax.fori_loop(0, (CH - 4) // 4, group, 0)
    g_wait(2); s_start(2); s_drain(0)
    g_wait(3); s_start(3); s_drain(1)
    s_drain(2)
    s_drain(3)
    plsc.subcore_barrier()
    pltpu.sync_copy(acc.at[pl.ds(s * RPT, RPT)],
                    out_hbm.at[c, pl.ds(s * RPT, RPT)])


@functools.cache
def _sc_segment():
    return pl.kernel(
        _sc_segment_body,
        out_type=jax.ShapeDtypeStruct((NC, NP, D), jnp.float32),
        mesh=plsc.VectorSubcoreMesh(core_axis_name="c", subcore_axis_name="s",
                                    num_cores=NC, num_subcores=NS),
        scratch_types=[
            pltpu.VMEM((CH // 2, 2 * K), jnp.int32),
            pltpu.VMEM((NB, K), jnp.int32),
            pltpu.VMEM((NB, K), jnp.int32),
            pltpu.VMEM((NB, K, D), jnp.float32),
            pltpu.VMEM_SHARED((NP, D), jnp.float32),
            pltpu.SemaphoreType.DMA,
            pltpu.SemaphoreType.DMA,
            pltpu.SemaphoreType.DMA,
        ],
    )


def _mlp_body(p0_ref, p1_ref, w0_ref, b0_ref, w1_ref, b1_ref, o_ref):
    i = pl.program_id(0)
    t = p0_ref[0] + p1_ref[0]
    a = jnp.dot(t, w0_ref[...], preferred_element_type=jnp.float32) + b0_ref[...]
    a = jnp.maximum(a, 0.0)
    a = jnp.dot(a, w1_ref[...], preferred_element_type=jnp.float32) + b1_ref[...]
    a = jnp.maximum(a, 0.0)
    rows = i * BLK + lax.broadcasted_iota(jnp.int32, (BLK, 1), 0)
    o_ref[...] = jnp.where(rows < N, a, 0.0)


def _tc_mlp(p, w0, b0, w1, b1):
    row_spec = pl.BlockSpec((BLK, D), lambda i: (i, 0))
    full = lambda shape: pl.BlockSpec(shape, lambda i: (0, 0))
    return pl.pallas_call(
        _mlp_body,
        grid=(NBLK,),
        in_specs=[pl.BlockSpec((1, BLK, D), lambda i: (0, i, 0)),
                  pl.BlockSpec((1, BLK, D), lambda i: (1, i, 0)),
                  full((D, D)), full((1, D)), full((D, D)), full((1, D))],
        out_specs=row_spec,
        out_shape=jax.ShapeDtypeStruct((NP, D), jnp.float32),
    )(p, p, w0, b0, w1, b1)


def _final_body(p0_ref, p1_ref, batch_ref, w0_ref, b0_ref, w1_ref,
                b1_ref, wout_ref, bout_ref, o_ref, pooled):
    i = pl.program_id(0)
    t = p0_ref[0] + p1_ref[0]
    a = jnp.dot(t, w0_ref[...], preferred_element_type=jnp.float32) + b0_ref[...]
    a = jnp.maximum(a, 0.0)
    a = jnp.dot(a, w1_ref[...], preferred_element_type=jnp.float32) + b1_ref[...]
    a = jnp.maximum(a, 0.0)

    @pl.when(i == 0)
    def _():
        pooled[...] = jnp.full((G, D), -jnp.inf, dtype=jnp.float32)

    b = batch_ref[...]
    for g in range(G):
        v = jnp.where(b == g, a, -jnp.inf).max(axis=0)
        pooled[g, :] = jnp.maximum(pooled[g, :], v)

    @pl.when(i == NBLK - 1)
    def _():
        o_ref[...] = (
            jnp.dot(pooled[...], wout_ref[...], preferred_element_type=jnp.float32)
            + bout_ref[...])


def _tc_final(p, batch2d, w0, b0, w1, b1, wout, bout):
    full = lambda shape: pl.BlockSpec(shape, lambda i: (0, 0))
    return pl.pallas_call(
        _final_body,
        grid=(NBLK,),
        in_specs=[pl.BlockSpec((1, BLK, D), lambda i: (0, i, 0)),
                  pl.BlockSpec((1, BLK, D), lambda i: (1, i, 0)),
                  pl.BlockSpec((BLK, 1), lambda i: (i, 0)),
                  full((D, D)), full((1, D)), full((D, D)), full((1, D)),
                  full((D, C)), full((1, C))],
        out_specs=full((G, C)),
        out_shape=jax.ShapeDtypeStruct((G, C), jnp.float32),
        scratch_shapes=[pltpu.VMEM((G, D), jnp.float32)],
    )(p, p, batch2d, w0, b0, w1, b1, wout, bout)


def kernel(x, edge_index, batch, W00, b00, W01, b01, W10, b10, W11, b11,
           W20, b20, W21, b21, W30, b30, W31, b31, Wout, bout):
    xp = jnp.zeros((NP, D), jnp.float32).at[:N].set(x)
    npad = EP - E
    srcp = jnp.concatenate(
        [edge_index[0], N + (jnp.arange(npad, dtype=jnp.int32) % (NP - N))])
    dstp = jnp.concatenate(
        [edge_index[1], (jnp.arange(npad, dtype=jnp.int32) * 131) % N])
    pidx = ((dstp << 14) | srcp).reshape(NW, CH // 2, 2 * K)
    batch2d = jnp.full((NP, 1), G, jnp.int32).at[:N, 0].set(batch)

    ws = [(W00, b00.reshape(1, D), W01, b01.reshape(1, D)),
          (W10, b10.reshape(1, D), W11, b11.reshape(1, D)),
          (W20, b20.reshape(1, D), W21, b21.reshape(1, D)),
          (W30, b30.reshape(1, D), W31, b31.reshape(1, D))]

    zrows = jnp.zeros((RPT, D), jnp.float32)
    seg = _sc_segment()
    h = xp
    for l in range(3):
        p = seg(h, zrows, pidx)
        h = _tc_mlp(p, *ws[l])
    p = seg(h, zrows, pidx)
    return _tc_final(p, batch2d, *ws[3], Wout, bout.reshape(1, C))


# speedup vs baseline: 3.9865x; 1.0158x over previous
import functools

import jax
import jax.numpy as jnp
from jax import lax
from jax.experimental import pallas as pl
from jax.experimental.pallas import tpu as pltpu
import jax.experimental.pallas.tpu_sc as plsc

N = 10000
D = 128
E = 320000
G = 16
C = 2

NC = 2
NS = 16
NW = NC * NS
K = 64
CH = 160
EP = NW * K * CH
NB = 4
NP = 10240
RPT = NP // NS

BLK = 2048
NBLK = NP // BLK


def _sc_segment_body(h_hbm, zrows_hbm, pidx_hbm, out_hbm, pidx, sbuf, dbuf,
                     rows, acc, sem_g, sem_s, sem_i):
    c = lax.axis_index("c")
    s = lax.axis_index("s")
    w = c * NS + s
    pltpu.sync_copy(pidx_hbm.at[w], pidx)
    @pl.when(c == 0)
    def _():
        pltpu.async_copy(h_hbm.at[pl.ds(s * RPT, RPT)],
                         acc.at[pl.ds(s * RPT, RPT)], sem_i)

    @pl.when(c == 1)
    def _():
        pltpu.async_copy(zrows_hbm, acc.at[pl.ds(s * RPT, RPT)], sem_i)

    def unpack(row, half, b):
        for q in range(K // 16):
            v = pidx[row, pl.ds(half * K + q * 16, 16)]
            sbuf[b, pl.ds(q * 16, 16)] = v & 0x3FFF
            dbuf[b, pl.ds(q * 16, 16)] = v >> 14

    def g_start(b):
        pltpu.async_copy(h_hbm.at[sbuf.at[b]], rows.at[b], sem_g)

    def g_wait(b):
        pltpu.make_async_copy(h_hbm.at[pl.ds(0, K)], rows.at[b], sem_g).wait()

    def s_start(b):
        pltpu.async_copy(rows.at[b], acc.at[dbuf.at[b]], sem_s, add=True)

    def s_drain(b):
        pltpu.make_async_copy(rows.at[b], out_hbm.at[c, pl.ds(0, K)],
                              sem_s).wait()

    unpack(0, 0, 0); g_start(0)
    unpack(0, 1, 1); g_start(1)
    unpack(1, 0, 2); g_start(2)
    unpack(1, 1, 3); g_start(3)
    pltpu.make_async_copy(zrows_hbm, acc.at[pl.ds(s * RPT, RPT)], sem_i).wait()
    plsc.subcore_barrier()
    g_wait(0); s_start(0)
    g_wait(1); s_start(1)

    def group(g, carry):
        for b in range(4):
            bj = (b + 2) % 4
            g_wait(bj)
            s_start(bj)
            s_drain(b)
            unpack(2 * g + 2 + b // 2, b % 2, b)
            g_start(b)
        return carry

    lax.fori_loop(0, (CH - 4) // 4, group, 0)
    g_wait(2); s_start(2); s_drain(0)
    g_wait(3); s_start(3); s_drain(1)
    s_drain(2)
    s_drain(3)
    plsc.subcore_barrier()
    pltpu.sync_copy(acc.at[pl.ds(s * RPT, RPT)],
                    out_hbm.at[c, pl.ds(s * RPT, RPT)])


@functools.cache
def _sc_segment():
    return pl.kernel(
        _sc_segment_body,
        out_type=jax.ShapeDtypeStruct((NC, NP, D), jnp.float32),
        mesh=plsc.VectorSubcoreMesh(core_axis_name="c", subcore_axis_name="s",
                                    num_cores=NC, num_subcores=NS),
        scratch_types=[
            pltpu.VMEM((CH // 2, 2 * K), jnp.int32),
            pltpu.VMEM((NB, K), jnp.int32),
            pltpu.VMEM((NB, K), jnp.int32),
            pltpu.VMEM((NB, K, D), jnp.float32),
            pltpu.VMEM_SHARED((NP, D), jnp.float32),
            pltpu.SemaphoreType.DMA,
            pltpu.SemaphoreType.DMA,
            pltpu.SemaphoreType.DMA,
        ],
    )


def _mlp_body(p0_ref, p1_ref, w0_ref, b0_ref, w1_ref, b1_ref, o_ref):
    i = pl.program_id(0)
    t = p0_ref[0] + p1_ref[0]
    a = jnp.dot(t, w0_ref[...], preferred_element_type=jnp.float32) + b0_ref[...]
    a = jnp.maximum(a, 0.0)
    a = jnp.dot(a, w1_ref[...], preferred_element_type=jnp.float32) + b1_ref[...]
    a = jnp.maximum(a, 0.0)
    rows = i * BLK + lax.broadcasted_iota(jnp.int32, (BLK, 1), 0)
    o_ref[...] = jnp.where(rows < N, a, 0.0)


def _tc_mlp(p, w0, b0, w1, b1):
    row_spec = pl.BlockSpec((BLK, D), lambda i: (i, 0))
    full = lambda shape: pl.BlockSpec(shape, lambda i: (0, 0))
    return pl.pallas_call(
        _mlp_body,
        grid=(NBLK,),
        in_specs=[pl.BlockSpec((1, BLK, D), lambda i: (0, i, 0)),
                  pl.BlockSpec((1, BLK, D), lambda i: (1, i, 0)),
                  full((D, D)), full((1, D)), full((D, D)), full((1, D))],
        out_specs=row_spec,
        out_shape=jax.ShapeDtypeStruct((NP, D), jnp.float32),
    )(p, p, w0, b0, w1, b1)


def _final_body(p0_ref, p1_ref, batch_ref, w0_ref, b0_ref, w1_ref,
                b1_ref, wout_ref, bout_ref, o_ref, pooled):
    i = pl.program_id(0)
    t = p0_ref[0] + p1_ref[0]
    a = jnp.dot(t, w0_ref[...], preferred_element_type=jnp.float32) + b0_ref[...]
    a = jnp.maximum(a, 0.0)
    a = jnp.dot(a, w1_ref[...], preferred_element_type=jnp.float32) + b1_ref[...]
    a = jnp.maximum(a, 0.0)

    @pl.when(i == 0)
    def _():
        pooled[...] = jnp.full((G, D), -jnp.inf, dtype=jnp.float32)

    b = batch_ref[...]
    for g in range(G):
        v = jnp.where(b == g, a, -jnp.inf).max(axis=0)
        pooled[g, :] = jnp.maximum(pooled[g, :], v)

    @pl.when(i == NBLK - 1)
    def _():
        o_ref[...] = (
            jnp.dot(pooled[...], wout_ref[...], preferred_element_type=jnp.float32)
            + bout_ref[...])


def _tc_final(p, batch2d, w0, b0, w1, b1, wout, bout):
    full = lambda shape: pl.BlockSpec(shape, lambda i: (0, 0))
    return pl.pallas_call(
        _final_body,
        grid=(NBLK,),
        in_specs=[pl.BlockSpec((1, BLK, D), lambda i: (0, i, 0)),
                  pl.BlockSpec((1, BLK, D), lambda i: (1, i, 0)),
                  pl.BlockSpec((BLK, 1), lambda i: (i, 0)),
                  full((D, D)), full((1, D)), full((D, D)), full((1, D)),
                  full((D, C)), full((1, C))],
        out_specs=full((G, C)),
        out_shape=jax.ShapeDtypeStruct((G, C), jnp.float32),
        scratch_shapes=[pltpu.VMEM((G, D), jnp.float32)],
    )(p, p, batch2d, w0, b0, w1, b1, wout, bout)


def kernel(x, edge_index, batch, W00, b00, W01, b01, W10, b10, W11, b11,
           W20, b20, W21, b21, W30, b30, W31, b31, Wout, bout):
    xp = jnp.zeros((NP, D), jnp.float32).at[:N].set(x)
    npad = EP - E
    srcp = jnp.concatenate(
        [edge_index[0], N + (jnp.arange(npad, dtype=jnp.int32) % (NP - N))])
    dstp = jnp.concatenate(
        [edge_index[1], (jnp.arange(npad, dtype=jnp.int32) * 131) % N])
    pidx = ((dstp << 14) | srcp).reshape(NW, CH // 2, 2 * K)
    batch2d = jnp.full((NP, 1), G, jnp.int32).at[:N, 0].set(batch)

    ws = [(W00, b00.reshape(1, D), W01, b01.reshape(1, D)),
          (W10, b10.reshape(1, D), W11, b11.reshape(1, D)),
          (W20, b20.reshape(1, D), W21, b21.reshape(1, D)),
          (W30, b30.reshape(1, D), W31, b31.reshape(1, D))]

    zrows = jnp.zeros((RPT, D), jnp.float32)
    seg = _sc_segment()
    h = xp
    for l in range(3):
        p = seg(h, zrows, pidx)
        h = _tc_mlp(p, *ws[l])
    p = seg(h, zrows, pidx)
    return _tc_final(p, batch2d, *ws[3], Wout, bout.reshape(1, C))
